# Initial kernel scaffold; baseline (speedup 1.0000x reference)
import jax
import jax.numpy as jnp
from jax.experimental import pallas as pl


def kernel(x, tokens, pe):
    raise NotImplementedError("write your pallas kernel here")


# speedup vs baseline: 1.1454x; 1.1454x over previous
import functools

import jax
import jax.numpy as jnp
from jax import lax
from jax.experimental import pallas as pl
from jax.experimental.pallas import tpu as pltpu

SEP_ID = 102
W = 512
WIN = W + 16


def _prepass_body(tok_ref, pos_ref, base_ref, nch_ref, maxoff_ref, *, nrow, nblk, max_seq):
    mask = (tok_ref[...] == SEP_ID).astype(jnp.int32)
    within = mask
    shift = 1
    while shift < W:
        z = jnp.zeros((nrow, nblk, shift), jnp.int32)
        within = within + jnp.concatenate([z, within[:, :, :-shift]], axis=2)
        shift *= 2
    nsep = within[:, :, W - 1:W]
    cinc = nsep
    shift = 1
    while shift < nblk:
        z = jnp.zeros((nrow, shift, 1), jnp.int32)
        cinc = cinc + jnp.concatenate([z, cinc[:, :-shift, :]], axis=1)
        shift *= 2
    carry = cinc - nsep
    pos_ref[...] = within + carry

    base = jnp.clip(carry, 0, max_seq - WIN)
    base = base - base % 8
    pmax = jnp.minimum(carry + nsep, max_seq - 1)
    maxoff = pmax - base
    base_ref[...] = base
    nch_ref[...] = maxoff // 8 + 1
    maxoff_ref[...] = maxoff


def _main_body(base_sref, nch_sref, maxoff_sref, pos_ref, x_ref, pe_ref, out_ref,
               window, sem, *, max_seq):
    i = pl.program_id(0)
    base = base_sref[i]
    nch = nch_sref[i]
    maxoff = maxoff_sref[i]

    def fetch(j, _):
        cp = pltpu.make_async_copy(
            pe_ref.at[pl.ds(pl.multiple_of(base + 8 * j, 8), 8), :],
            window.at[pl.ds(8 * j, 8), :],
            sem,
        )
        cp.start()
        cp.wait()
        return 0

    lax.fori_loop(0, nch, fetch, 0)

    off = jnp.clip(pos_ref[0], 0, max_seq - 1) - base
    x = x_ref[0]
    out_ref[0] = x + window[pl.ds(0, 1), :]

    def blend(d, _):
        row = window[pl.ds(d, 1), :]
        out_ref[0] = jnp.where(off == d, x + row, out_ref[0])
        return 0

    lax.fori_loop(1, maxoff + 1, blend, 0)


def kernel(x, tokens, pe):
    B, L, D = x.shape
    max_seq = pe.shape[1]
    nblk = L // W
    nb = B * nblk

    tok3 = tokens.reshape(B, nblk, W)
    prepass = pl.pallas_call(
        functools.partial(_prepass_body, nrow=B, nblk=nblk, max_seq=max_seq),
        out_shape=(
            jax.ShapeDtypeStruct((B, nblk, W), jnp.int32),
            jax.ShapeDtypeStruct((B, nblk, 1), jnp.int32),
            jax.ShapeDtypeStruct((B, nblk, 1), jnp.int32),
            jax.ShapeDtypeStruct((B, nblk, 1), jnp.int32),
        ),
    )
    pos, base, nch, maxoff = prepass(tok3)

    grid_spec = pltpu.PrefetchScalarGridSpec(
        num_scalar_prefetch=3,
        grid=(nb,),
        in_specs=[
            pl.BlockSpec((1, W, 1), lambda i, *_: (i, 0, 0)),
            pl.BlockSpec((1, W, D), lambda i, *_: (i, 0, 0)),
            pl.BlockSpec(memory_space=pltpu.MemorySpace.HBM),
        ],
        out_specs=pl.BlockSpec((1, W, D), lambda i, *_: (i, 0, 0)),
        scratch_shapes=[
            pltpu.VMEM((WIN, D), jnp.float32),
            pltpu.SemaphoreType.DMA,
        ],
    )
    main = pl.pallas_call(
        functools.partial(_main_body, max_seq=max_seq),
        grid_spec=grid_spec,
        out_shape=jax.ShapeDtypeStruct((nb, W, D), jnp.float32),
        compiler_params=pltpu.CompilerParams(
            dimension_semantics=("arbitrary",),
        ),
    )
    out = main(
        base.reshape(nb), nch.reshape(nb), maxoff.reshape(nb),
        pos.reshape(nb, W, 1), x.reshape(nb, W, D), pe[0],
    )
    return out.reshape(B, L, D)


# speedup vs baseline: 2.4687x; 2.1553x over previous
import functools

import jax
import jax.numpy as jnp
from jax import lax
from jax.experimental import pallas as pl
from jax.experimental.pallas import tpu as pltpu

SEP_ID = 102
W = 512
WIN = W + 16


def _prepass_body(tok_ref, pos_ref, base8_ref, nch_ref, minoff_ref, maxoff_ref,
                  *, nrow, nblk, max_seq):
    mask = (tok_ref[...] == SEP_ID).astype(jnp.int32)
    within = mask
    shift = 1
    while shift < W:
        z = jnp.zeros((nrow, nblk, shift), jnp.int32)
        within = within + jnp.concatenate([z, within[:, :, :-shift]], axis=2)
        shift *= 2
    nsep = within[:, :, W - 1:W]
    cinc = nsep
    shift = 1
    while shift < nblk:
        z = jnp.zeros((nrow, shift, 1), jnp.int32)
        cinc = cinc + jnp.concatenate([z, cinc[:, :-shift, :]], axis=1)
        shift *= 2
    carry = cinc - nsep
    pos_ref[...] = within + carry

    base = jnp.clip(carry, 0, max_seq - WIN)
    base = base - base % 8
    pmax = jnp.minimum(carry + nsep, max_seq - 1)
    maxoff = pmax - base
    base8_ref[...] = base // 8
    nch_ref[...] = maxoff // 8 + 1
    minoff_ref[...] = jnp.clip(carry, 0, max_seq - 1) - base
    maxoff_ref[...] = maxoff


def _main_body(base8_s, nch_s, minoff_s, maxoff_s, pos_ref, x_ref, peblk_ref,
               pe_ref, out_ref, window, sem, *, max_seq):
    i = pl.program_id(0)
    base = base8_s[i] * 8
    window[pl.ds(0, 8), :] = peblk_ref[...]
    nch = nch_s[i]

    @pl.when(nch > 1)
    def _fetch_rest():
        def fetch(j, _):
            cp = pltpu.make_async_copy(
                pe_ref.at[pl.ds(pl.multiple_of(base + 8 * j, 8), 8), :],
                window.at[pl.ds(8 * j, 8), :],
                sem,
            )
            cp.start()
            cp.wait()
            return 0

        lax.fori_loop(1, nch, fetch, 0)

    off = jnp.clip(pos_ref[0], 0, max_seq - 1) - base
    x = x_ref[0]
    mo = minoff_s[i]
    out_ref[0] = x + window[pl.ds(mo, 1), :]

    def blend(d, _):
        row = window[pl.ds(d, 1), :]
        out_ref[0] = jnp.where(off == d, x + row, out_ref[0])
        return 0

    lax.fori_loop(mo + 1, maxoff_s[i] + 1, blend, 0)


def kernel(x, tokens, pe):
    B, L, D = x.shape
    max_seq = pe.shape[1]
    nblk = L // W
    nb = B * nblk

    tok3 = tokens.reshape(B, nblk, W)
    prepass = pl.pallas_call(
        functools.partial(_prepass_body, nrow=B, nblk=nblk, max_seq=max_seq),
        out_shape=(
            jax.ShapeDtypeStruct((B, nblk, W), jnp.int32),
            jax.ShapeDtypeStruct((B, nblk, 1), jnp.int32),
            jax.ShapeDtypeStruct((B, nblk, 1), jnp.int32),
            jax.ShapeDtypeStruct((B, nblk, 1), jnp.int32),
            jax.ShapeDtypeStruct((B, nblk, 1), jnp.int32),
        ),
    )
    pos, base8, nch, minoff, maxoff = prepass(tok3)

    grid_spec = pltpu.PrefetchScalarGridSpec(
        num_scalar_prefetch=4,
        grid=(nb,),
        in_specs=[
            pl.BlockSpec((1, W, 1), lambda i, *_: (i, 0, 0)),
            pl.BlockSpec((1, W, D), lambda i, *_: (i, 0, 0)),
            pl.BlockSpec((8, D), lambda i, base8, *_: (base8[i], 0)),
            pl.BlockSpec(memory_space=pltpu.MemorySpace.HBM),
        ],
        out_specs=pl.BlockSpec((1, W, D), lambda i, *_: (i, 0, 0)),
        scratch_shapes=[
            pltpu.VMEM((WIN, D), jnp.float32),
            pltpu.SemaphoreType.DMA,
        ],
    )
    main = pl.pallas_call(
        functools.partial(_main_body, max_seq=max_seq),
        grid_spec=grid_spec,
        out_shape=jax.ShapeDtypeStruct((nb, W, D), jnp.float32),
        compiler_params=pltpu.CompilerParams(
            dimension_semantics=("arbitrary",),
        ),
    )
    out = main(
        base8.reshape(nb), nch.reshape(nb), minoff.reshape(nb),
        maxoff.reshape(nb),
        pos.reshape(nb, W, 1), x.reshape(nb, W, D), pe[0], pe[0],
    )
    return out.reshape(B, L, D)


# speedup vs baseline: 2.6769x; 1.0844x over previous
import functools

import jax
import jax.numpy as jnp
from jax import lax
from jax.experimental import pallas as pl
from jax.experimental.pallas import tpu as pltpu

SEP_ID = 102
W = 1024
WIN = W + 16


def _prepass_body(tok_ref, pos_ref, base8_ref, nch_ref, minoff_ref, maxoff_ref,
                  *, nrow, nblk, max_seq):
    mask = (tok_ref[...] == SEP_ID).astype(jnp.int32)
    within = mask
    shift = 1
    while shift < W:
        z = jnp.zeros((nrow, nblk, shift), jnp.int32)
        within = within + jnp.concatenate([z, within[:, :, :-shift]], axis=2)
        shift *= 2
    nsep = within[:, :, W - 1:W]
    cinc = nsep
    shift = 1
    while shift < nblk:
        z = jnp.zeros((nrow, shift, 1), jnp.int32)
        cinc = cinc + jnp.concatenate([z, cinc[:, :-shift, :]], axis=1)
        shift *= 2
    carry = cinc - nsep
    pos_ref[...] = within + carry

    base = jnp.clip(carry, 0, max_seq - WIN)
    base = base - base % 8
    pmax = jnp.minimum(carry + nsep, max_seq - 1)
    maxoff = pmax - base
    base8_ref[...] = base // 8
    nch_ref[...] = maxoff // 8 + 1
    minoff_ref[...] = jnp.clip(carry, 0, max_seq - 1) - base
    maxoff_ref[...] = maxoff


def _main_body(base8_s, nch_s, minoff_s, maxoff_s, pos_ref, x_ref, peblk_ref,
               pe_ref, out_ref, window, sem, *, max_seq):
    i = pl.program_id(0)
    base = base8_s[i] * 8
    window[pl.ds(0, 8), :] = peblk_ref[...]
    nch = nch_s[i]

    @pl.when(nch > 1)
    def _fetch_rest():
        def fetch(j, _):
            cp = pltpu.make_async_copy(
                pe_ref.at[pl.ds(pl.multiple_of(base + 8 * j, 8), 8), :],
                window.at[pl.ds(8 * j, 8), :],
                sem,
            )
            cp.start()
            cp.wait()
            return 0

        lax.fori_loop(1, nch, fetch, 0)

    off = jnp.clip(pos_ref[0], 0, max_seq - 1) - base
    x = x_ref[0]
    mo = minoff_s[i]
    out_ref[0] = x + window[pl.ds(mo, 1), :]

    def blend(d, _):
        row = window[pl.ds(d, 1), :]
        out_ref[0] = jnp.where(off == d, x + row, out_ref[0])
        return 0

    lax.fori_loop(mo + 1, maxoff_s[i] + 1, blend, 0)


def kernel(x, tokens, pe):
    B, L, D = x.shape
    max_seq = pe.shape[1]
    nblk = L // W
    nb = B * nblk

    tok3 = tokens.reshape(B, nblk, W)
    prepass = pl.pallas_call(
        functools.partial(_prepass_body, nrow=B, nblk=nblk, max_seq=max_seq),
        out_shape=(
            jax.ShapeDtypeStruct((B, nblk, W), jnp.int32),
            jax.ShapeDtypeStruct((B, nblk, 1), jnp.int32),
            jax.ShapeDtypeStruct((B, nblk, 1), jnp.int32),
            jax.ShapeDtypeStruct((B, nblk, 1), jnp.int32),
            jax.ShapeDtypeStruct((B, nblk, 1), jnp.int32),
        ),
    )
    pos, base8, nch, minoff, maxoff = prepass(tok3)

    grid_spec = pltpu.PrefetchScalarGridSpec(
        num_scalar_prefetch=4,
        grid=(nb,),
        in_specs=[
            pl.BlockSpec((1, W, 1), lambda i, *_: (i, 0, 0)),
            pl.BlockSpec((1, W, D), lambda i, *_: (i, 0, 0)),
            pl.BlockSpec((8, D), lambda i, base8, *_: (base8[i], 0)),
            pl.BlockSpec(memory_space=pltpu.MemorySpace.HBM),
        ],
        out_specs=pl.BlockSpec((1, W, D), lambda i, *_: (i, 0, 0)),
        scratch_shapes=[
            pltpu.VMEM((WIN, D), jnp.float32),
            pltpu.SemaphoreType.DMA,
        ],
    )
    main = pl.pallas_call(
        functools.partial(_main_body, max_seq=max_seq),
        grid_spec=grid_spec,
        out_shape=jax.ShapeDtypeStruct((nb, W, D), jnp.float32),
        compiler_params=pltpu.CompilerParams(
            dimension_semantics=("arbitrary",),
        ),
    )
    out = main(
        base8.reshape(nb), nch.reshape(nb), minoff.reshape(nb),
        maxoff.reshape(nb),
        pos.reshape(nb, W, 1), x.reshape(nb, W, D), pe[0], pe[0],
    )
    return out.reshape(B, L, D)
